# baseline (speedup 1.0000x reference)
import functools

import jax
import jax.numpy as jnp
from jax import lax
from jax.experimental import pallas as pl
from jax.experimental.pallas import tpu as pltpu
from jax.experimental.pallas import tpu_sc as plsc

NC = 2
NS = 16
NW = NC * NS
C = 128
D = 128


def _matmul2_body(x_ref, wa_ref, wb_ref, a_ref, b_ref):
    x = x_ref[...]
    dn = (((1,), (1,)), ((), ()))
    a_ref[...] = lax.dot_general(x, wa_ref[...], dn,
                                 preferred_element_type=jnp.float32)
    b_ref[...] = lax.dot_general(x, wb_ref[...], dn,
                                 preferred_element_type=jnp.float32)


def _tc_dual_matmul(x, wa, wb, block_rows):
    n = x.shape[0]
    grid = n // block_rows
    return pl.pallas_call(
        _matmul2_body,
        grid=(grid,),
        in_specs=[
            pl.BlockSpec((block_rows, D), lambda i: (i, 0)),
            pl.BlockSpec((D, D), lambda i: (0, 0)),
            pl.BlockSpec((D, D), lambda i: (0, 0)),
        ],
        out_specs=[
            pl.BlockSpec((block_rows, D), lambda i: (i, 0)),
            pl.BlockSpec((block_rows, D), lambda i: (i, 0)),
        ],
        out_shape=[
            jax.ShapeDtypeStruct((n, D), jnp.float32),
            jax.ShapeDtypeStruct((n, D), jnp.float32),
        ],
    )(x, wa, wb)


def _combine2_body(s_ref, p0_ref, p1_ref, b_ref, wa_ref, wb_ref, a_ref,
                   bo_ref):
    h = s_ref[...] + p0_ref[...] + p1_ref[...] + b_ref[...]
    dn = (((1,), (1,)), ((), ()))
    a_ref[...] = lax.dot_general(h, wa_ref[...], dn,
                                 preferred_element_type=jnp.float32)
    bo_ref[...] = lax.dot_general(h, wb_ref[...], dn,
                                  preferred_element_type=jnp.float32)


def _tc_combine_matmul(s, p0, p1, bias, wa, wb, block_rows):
    n = s.shape[0]
    grid = n // block_rows
    row_spec = pl.BlockSpec((block_rows, D), lambda i: (i, 0))
    w_spec = pl.BlockSpec((D, D), lambda i: (0, 0))
    return pl.pallas_call(
        _combine2_body,
        grid=(grid,),
        in_specs=[row_spec, row_spec, row_spec,
                  pl.BlockSpec((D,), lambda i: (0,)), w_spec, w_spec],
        out_specs=[row_spec, row_spec],
        out_shape=[
            jax.ShapeDtypeStruct((n, D), jnp.float32),
            jax.ShapeDtypeStruct((n, D), jnp.float32),
        ],
    )(s, p0, p1, bias, wa, wb)


def _final_body(s_ref, p0_ref, p1_ref, b_ref, o_ref):
    o_ref[...] = s_ref[...] + p0_ref[...] + p1_ref[...] + b_ref[...]


def _tc_final(s, p0, p1, bias, block_rows):
    n = s.shape[0]
    grid = n // block_rows
    row_spec = pl.BlockSpec((block_rows, D), lambda i: (i, 0))
    return pl.pallas_call(
        _final_body,
        grid=(grid,),
        in_specs=[row_spec, row_spec, row_spec,
                  pl.BlockSpec((D,), lambda i: (0,))],
        out_specs=row_spec,
        out_shape=jax.ShapeDtypeStruct((n, D), jnp.float32),
    )(s, p0, p1, bias)


def _make_sc_aggregate(n_pad, e_pad, with_deg):
    chunks_total = e_pad // C
    row_chunks = chunks_total // NW
    deg_chunks = chunks_total // NS
    rows_per_tile = n_pad // NS
    blocks_per_tile = rows_per_tile // C

    mesh = plsc.VectorSubcoreMesh(core_axis_name="c", subcore_axis_name="s")

    out_type = [jax.ShapeDtypeStruct((NC, n_pad, D), jnp.float32)]
    if with_deg:
        out_type.append(jax.ShapeDtypeStruct((NC, n_pad), jnp.float32))

    scratch = [
        pltpu.VMEM((C,), jnp.int32),
        pltpu.VMEM((C,), jnp.int32),
        pltpu.VMEM((C, D), jnp.float32),
        pltpu.VMEM((C,), jnp.float32),
        pltpu.VMEM((rows_per_tile,), jnp.float32),
        pltpu.VMEM_SHARED((n_pad, D), jnp.float32),
        pltpu.VMEM_SHARED((n_pad,), jnp.float32),
        pltpu.SemaphoreType.DMA,
    ]

    @functools.partial(
        pl.kernel, mesh=mesh, scratch_types=scratch, out_type=out_type)
    def sc_kernel(y_hbm, src_hbm, dst_hbm, *rest):
        if with_deg:
            (p_hbm, iv_hbm,
             src_v, dst_v, rows_v, ones_v, inv_v, acc_s, deg_s, sem) = rest
        else:
            (inv_hbm, p_hbm,
             src_v, dst_v, rows_v, ones_v, inv_v, acc_s, deg_s, sem) = rest
        cid = lax.axis_index("c")
        sid = lax.axis_index("s")
        wid = cid * NS + sid
        my_row0 = sid * rows_per_tile
        zvec = jnp.zeros((16,), jnp.float32)
        ovec = jnp.ones((16,), jnp.float32)

        def zero_rows(i, _):
            for j in range(D // 16):
                rows_v[i, pl.ds(j * 16, 16)] = zvec
            return 0

        lax.fori_loop(0, C, zero_rows, 0)

        def zero_inv(i, _):
            inv_v[pl.ds(i * 16, 16)] = zvec
            return 0

        lax.fori_loop(0, rows_per_tile // 16, zero_inv, 0)

        for j in range(C // 16):
            ones_v[pl.ds(j * 16, 16)] = ovec

        for b in range(blocks_per_tile):
            pltpu.sync_copy(rows_v, acc_s.at[pl.ds(my_row0 + b * C, C)])
        if with_deg:
            pltpu.sync_copy(inv_v, deg_s.at[pl.ds(my_row0, rows_per_tile)])
        else:
            pltpu.sync_copy(inv_hbm.at[pl.ds(my_row0, rows_per_tile)],
                            inv_v)

        plsc.subcore_barrier()

        if with_deg:
            def deg_body(k, _):
                base = (sid * deg_chunks + k) * C
                pltpu.sync_copy(dst_hbm.at[pl.ds(base, C)], dst_v)
                pltpu.sync_copy(ones_v, deg_s.at[dst_v], add=True)
                return 0

            lax.fori_loop(0, deg_chunks, deg_body, 0)

        def chunk_body(k, _):
            base = (wid * row_chunks + k) * C
            pltpu.sync_copy(src_hbm.at[pl.ds(base, C)], src_v)
            pltpu.sync_copy(dst_hbm.at[pl.ds(base, C)], dst_v)
            pltpu.async_copy(y_hbm.at[src_v], rows_v, sem).wait()
            pltpu.sync_copy(rows_v, acc_s.at[dst_v], add=True)
            return 0

        lax.fori_loop(0, row_chunks, chunk_body, 0)

        plsc.subcore_barrier()

        if with_deg:
            pltpu.sync_copy(deg_s.at[pl.ds(my_row0, rows_per_tile)], inv_v)

            def invert(i, _):
                d = inv_v[pl.ds(i * 16, 16)]
                inv_v[pl.ds(i * 16, 16)] = ovec / jnp.maximum(d, ovec)
                return 0

            lax.fori_loop(0, rows_per_tile // 16, invert, 0)
            pltpu.sync_copy(inv_v,
                            iv_hbm.at[cid, pl.ds(my_row0, rows_per_tile)])

        for b in range(blocks_per_tile):
            r0 = my_row0 + b * C
            pltpu.sync_copy(acc_s.at[pl.ds(r0, C)], rows_v)

            def scale16(g, _):
                iv = inv_v[pl.ds(b * C + g * 16, 16)]
                for k in range(16):
                    s = zvec + iv[k]
                    row = g * 16 + k
                    for j in range(D // 16):
                        rows_v[row, pl.ds(j * 16, 16)] = (
                            rows_v[row, pl.ds(j * 16, 16)] * s)
                return 0

            lax.fori_loop(0, C // 16, scale16, 0)
            pltpu.sync_copy(rows_v, p_hbm.at[cid, pl.ds(r0, C)])

    return sc_kernel


def kernel(features, edge_index, W_self1, W_neigh1, b1, W_self2, W_neigh2,
           b2):
    n = features.shape[0]
    e = edge_index.shape[1]

    n_pad = -(-(n + 1) // (NS * C)) * (NS * C)
    e_pad = -(-e // (NW * C * 8)) * (NW * C * 8)

    src = edge_index[0]
    dst = edge_index[1]
    if e_pad > e:
        pad = e_pad - e
        src = jnp.concatenate([src, jnp.zeros((pad,), jnp.int32)])
        dst = jnp.concatenate([dst, jnp.full((pad,), n, jnp.int32)])

    block_rows = 2000

    sc_agg_deg = _make_sc_aggregate(n_pad, e_pad, True)
    sc_agg = _make_sc_aggregate(n_pad, e_pad, False)

    y1, s1 = _tc_dual_matmul(features, W_neigh1, W_self1, block_rows)
    a1, inv = sc_agg_deg(y1, src, dst)
    y2, s2 = _tc_combine_matmul(s1, a1[0, :n], a1[1, :n], b1, W_neigh2,
                                W_self2, block_rows)
    a2 = sc_agg(y2, src, dst, inv[0])
    if isinstance(a2, (tuple, list)):
        a2 = a2[0]
    out = _tc_final(s2, a2[0, :n], a2[1, :n], b2, block_rows)
    return out


# speedup vs baseline: 2.0221x; 2.0221x over previous
import functools

import jax
import jax.numpy as jnp
from jax import lax
from jax.experimental import pallas as pl
from jax.experimental.pallas import tpu as pltpu
from jax.experimental.pallas import tpu_sc as plsc

NC = 2
NS = 16
NW = NC * NS
C = 128
D = 128


def _matmul2_body(x_ref, wa_ref, wb_ref, a_ref, b_ref):
    x = x_ref[...]
    dn = (((1,), (1,)), ((), ()))
    a_ref[...] = lax.dot_general(x, wa_ref[...], dn,
                                 preferred_element_type=jnp.float32)
    b_ref[...] = lax.dot_general(x, wb_ref[...], dn,
                                 preferred_element_type=jnp.float32)


def _tc_dual_matmul(x, wa, wb, block_rows):
    n = x.shape[0]
    grid = n // block_rows
    return pl.pallas_call(
        _matmul2_body,
        grid=(grid,),
        in_specs=[
            pl.BlockSpec((block_rows, D), lambda i: (i, 0)),
            pl.BlockSpec((D, D), lambda i: (0, 0)),
            pl.BlockSpec((D, D), lambda i: (0, 0)),
        ],
        out_specs=[
            pl.BlockSpec((block_rows, D), lambda i: (i, 0)),
            pl.BlockSpec((block_rows, D), lambda i: (i, 0)),
        ],
        out_shape=[
            jax.ShapeDtypeStruct((n, D), jnp.float32),
            jax.ShapeDtypeStruct((n, D), jnp.float32),
        ],
    )(x, wa, wb)


def _combine2_body(s_ref, p0_ref, p1_ref, b_ref, wa_ref, wb_ref, a_ref,
                   bo_ref):
    h = s_ref[...] + p0_ref[...] + p1_ref[...] + b_ref[...]
    dn = (((1,), (1,)), ((), ()))
    a_ref[...] = lax.dot_general(h, wa_ref[...], dn,
                                 preferred_element_type=jnp.float32)
    bo_ref[...] = lax.dot_general(h, wb_ref[...], dn,
                                  preferred_element_type=jnp.float32)


def _tc_combine_matmul(s, p0, p1, bias, wa, wb, block_rows):
    n = s.shape[0]
    grid = n // block_rows
    row_spec = pl.BlockSpec((block_rows, D), lambda i: (i, 0))
    w_spec = pl.BlockSpec((D, D), lambda i: (0, 0))
    return pl.pallas_call(
        _combine2_body,
        grid=(grid,),
        in_specs=[row_spec, row_spec, row_spec,
                  pl.BlockSpec((D,), lambda i: (0,)), w_spec, w_spec],
        out_specs=[row_spec, row_spec],
        out_shape=[
            jax.ShapeDtypeStruct((n, D), jnp.float32),
            jax.ShapeDtypeStruct((n, D), jnp.float32),
        ],
    )(s, p0, p1, bias, wa, wb)


def _final_body(s_ref, p0_ref, p1_ref, b_ref, o_ref):
    o_ref[...] = s_ref[...] + p0_ref[...] + p1_ref[...] + b_ref[...]


def _tc_final(s, p0, p1, bias, block_rows):
    n = s.shape[0]
    grid = n // block_rows
    row_spec = pl.BlockSpec((block_rows, D), lambda i: (i, 0))
    return pl.pallas_call(
        _final_body,
        grid=(grid,),
        in_specs=[row_spec, row_spec, row_spec,
                  pl.BlockSpec((D,), lambda i: (0,))],
        out_specs=row_spec,
        out_shape=jax.ShapeDtypeStruct((n, D), jnp.float32),
    )(s, p0, p1, bias)


def _make_sc_aggregate(n_pad, e_pad, with_deg):
    chunks_total = e_pad // C
    row_chunks = chunks_total // NW
    deg_chunks = chunks_total // NS
    rows_per_tile = n_pad // NS
    blocks_per_tile = rows_per_tile // C

    mesh = plsc.VectorSubcoreMesh(core_axis_name="c", subcore_axis_name="s")

    out_type = [jax.ShapeDtypeStruct((NC, n_pad, D), jnp.float32)]
    if with_deg:
        out_type.append(jax.ShapeDtypeStruct((NC, n_pad), jnp.float32))

    scratch = [
        pltpu.VMEM((C,), jnp.int32),
        pltpu.VMEM((C,), jnp.int32),
        pltpu.VMEM((C, D), jnp.float32),
        pltpu.VMEM((C,), jnp.float32),
        pltpu.VMEM((rows_per_tile,), jnp.float32),
        pltpu.VMEM_SHARED((n_pad, D), jnp.float32),
        pltpu.VMEM_SHARED((n_pad,), jnp.float32),
        pltpu.SemaphoreType.DMA,
    ]

    @functools.partial(
        pl.kernel, mesh=mesh, scratch_types=scratch, out_type=out_type)
    def sc_kernel(y_hbm, src_hbm, dst_hbm, *rest):
        if with_deg:
            (p_hbm, iv_hbm,
             src_v, dst_v, rows_v, ones_v, inv_v, acc_s, deg_s, sem) = rest
        else:
            (inv_hbm, p_hbm,
             src_v, dst_v, rows_v, ones_v, inv_v, acc_s, deg_s, sem) = rest
        cid = lax.axis_index("c")
        sid = lax.axis_index("s")
        wid = cid * NS + sid
        my_row0 = sid * rows_per_tile
        zvec = jnp.zeros((16,), jnp.float32)
        ovec = jnp.ones((16,), jnp.float32)

        def zero_rows(i, _):
            for j in range(D // 16):
                rows_v[i, pl.ds(j * 16, 16)] = zvec
            return 0

        lax.fori_loop(0, C, zero_rows, 0)

        def zero_inv(i, _):
            inv_v[pl.ds(i * 16, 16)] = zvec
            return 0

        lax.fori_loop(0, rows_per_tile // 16, zero_inv, 0)

        for j in range(C // 16):
            ones_v[pl.ds(j * 16, 16)] = ovec

        for b in range(blocks_per_tile):
            pltpu.sync_copy(rows_v, acc_s.at[pl.ds(my_row0 + b * C, C)])
        if with_deg:
            pltpu.sync_copy(inv_v, deg_s.at[pl.ds(my_row0, rows_per_tile)])
        else:
            pltpu.sync_copy(inv_hbm.at[pl.ds(my_row0, rows_per_tile)],
                            inv_v)

        plsc.subcore_barrier()

        if with_deg:
            def deg_body(k, _):
                base = (sid * deg_chunks + k) * C
                pltpu.sync_copy(dst_hbm.at[pl.ds(base, C)], dst_v)
                pltpu.sync_copy(ones_v, deg_s.at[dst_v], add=True)
                return 0

            lax.fori_loop(0, deg_chunks, deg_body, 0)

        def chunk_body(k, _):
            base = (wid * row_chunks + k) * C
            pltpu.sync_copy(src_hbm.at[pl.ds(base, C)], src_v)
            pltpu.sync_copy(dst_hbm.at[pl.ds(base, C)], dst_v)
            pltpu.async_copy(y_hbm.at[src_v], rows_v, sem).wait()
            pltpu.sync_copy(rows_v, acc_s.at[dst_v], add=True)
            return 0

        lax.fori_loop(0, row_chunks, chunk_body, 0)

        plsc.subcore_barrier()

        if with_deg:
            pltpu.sync_copy(deg_s.at[pl.ds(my_row0, rows_per_tile)], inv_v)

            def invert(i, _):
                d = inv_v[pl.ds(i * 16, 16)]
                inv_v[pl.ds(i * 16, 16)] = ovec / jnp.maximum(d, ovec)
                return 0

            lax.fori_loop(0, rows_per_tile // 16, invert, 0)
            pltpu.sync_copy(inv_v,
                            iv_hbm.at[cid, pl.ds(my_row0, rows_per_tile)])

        for b in range(blocks_per_tile):
            r0 = my_row0 + b * C
            pltpu.sync_copy(acc_s.at[pl.ds(r0, C)], rows_v)

            def scale16(g, _):
                iv = inv_v[pl.ds(b * C + g * 16, 16)]
                for k in range(16):
                    s = zvec + iv[k]
                    row = g * 16 + k
                    for j in range(D // 16):
                        rows_v[row, pl.ds(j * 16, 16)] = (
                            rows_v[row, pl.ds(j * 16, 16)] * s)
                return 0

            lax.fori_loop(0, C // 16, scale16, 0)
            pltpu.sync_copy(rows_v, p_hbm.at[cid, pl.ds(r0, C)])

    return sc_kernel


def kernel(features, edge_index, W_self1, W_neigh1, b1, W_self2, W_neigh2,
           b2):
    n = features.shape[0]
    e = edge_index.shape[1]

    n_pad = -(-(n + 1) // (NS * C)) * (NS * C)
    e_pad = -(-e // (NW * C * 8)) * (NW * C * 8)

    src = edge_index[0]
    dst = edge_index[1]
    if e_pad > e:
        pad = e_pad - e
        ar = jnp.arange(pad, dtype=jnp.int32)
        src = jnp.concatenate([src, ar % n])
        dst = jnp.concatenate([dst, n + ar % (n_pad - n)])

    block_rows = 2000

    sc_agg_deg = _make_sc_aggregate(n_pad, e_pad, True)
    sc_agg = _make_sc_aggregate(n_pad, e_pad, False)

    y1, s1 = _tc_dual_matmul(features, W_neigh1, W_self1, block_rows)
    a1, inv = sc_agg_deg(y1, src, dst)
    y2, s2 = _tc_combine_matmul(s1, a1[0, :n], a1[1, :n], b1, W_neigh2,
                                W_self2, block_rows)
    a2 = sc_agg(y2, src, dst, inv[0])
    if isinstance(a2, (tuple, list)):
        a2 = a2[0]
    out = _tc_final(s2, a2[0, :n], a2[1, :n], b2, block_rows)
    return out


# speedup vs baseline: 2.5934x; 1.2825x over previous
import functools

import jax
import jax.numpy as jnp
from jax import lax
from jax.experimental import pallas as pl
from jax.experimental.pallas import tpu as pltpu
from jax.experimental.pallas import tpu_sc as plsc

NC = 2
NS = 16
NW = NC * NS
C = 128
D = 128


def _matmul2_body(x_ref, wa_ref, wb_ref, a_ref, b_ref):
    x = x_ref[...]
    dn = (((1,), (1,)), ((), ()))
    a_ref[...] = lax.dot_general(x, wa_ref[...], dn,
                                 preferred_element_type=jnp.float32)
    b_ref[...] = lax.dot_general(x, wb_ref[...], dn,
                                 preferred_element_type=jnp.float32)


def _tc_dual_matmul(x, wa, wb, block_rows):
    n = x.shape[0]
    grid = n // block_rows
    return pl.pallas_call(
        _matmul2_body,
        grid=(grid,),
        in_specs=[
            pl.BlockSpec((block_rows, D), lambda i: (i, 0)),
            pl.BlockSpec((D, D), lambda i: (0, 0)),
            pl.BlockSpec((D, D), lambda i: (0, 0)),
        ],
        out_specs=[
            pl.BlockSpec((block_rows, D), lambda i: (i, 0)),
            pl.BlockSpec((block_rows, D), lambda i: (i, 0)),
        ],
        out_shape=[
            jax.ShapeDtypeStruct((n, D), jnp.float32),
            jax.ShapeDtypeStruct((n, D), jnp.float32),
        ],
    )(x, wa, wb)


def _combine2_body(s_ref, p0_ref, p1_ref, b_ref, wa_ref, wb_ref, a_ref,
                   bo_ref):
    h = s_ref[...] + p0_ref[...] + p1_ref[...] + b_ref[...]
    dn = (((1,), (1,)), ((), ()))
    a_ref[...] = lax.dot_general(h, wa_ref[...], dn,
                                 preferred_element_type=jnp.float32)
    bo_ref[...] = lax.dot_general(h, wb_ref[...], dn,
                                  preferred_element_type=jnp.float32)


def _tc_combine_matmul(s, p0, p1, bias, wa, wb, block_rows):
    n = s.shape[0]
    grid = n // block_rows
    row_spec = pl.BlockSpec((block_rows, D), lambda i: (i, 0))
    w_spec = pl.BlockSpec((D, D), lambda i: (0, 0))
    return pl.pallas_call(
        _combine2_body,
        grid=(grid,),
        in_specs=[row_spec, row_spec, row_spec,
                  pl.BlockSpec((D,), lambda i: (0,)), w_spec, w_spec],
        out_specs=[row_spec, row_spec],
        out_shape=[
            jax.ShapeDtypeStruct((n, D), jnp.float32),
            jax.ShapeDtypeStruct((n, D), jnp.float32),
        ],
    )(s, p0, p1, bias, wa, wb)


def _final_body(s_ref, p0_ref, p1_ref, b_ref, o_ref):
    o_ref[...] = s_ref[...] + p0_ref[...] + p1_ref[...] + b_ref[...]


def _tc_final(s, p0, p1, bias, block_rows):
    n = s.shape[0]
    grid = n // block_rows
    row_spec = pl.BlockSpec((block_rows, D), lambda i: (i, 0))
    return pl.pallas_call(
        _final_body,
        grid=(grid,),
        in_specs=[row_spec, row_spec, row_spec,
                  pl.BlockSpec((D,), lambda i: (0,))],
        out_specs=row_spec,
        out_shape=jax.ShapeDtypeStruct((n, D), jnp.float32),
    )(s, p0, p1, bias)


def _make_sc_aggregate(n_pad, e_pad, with_deg):
    chunks_total = e_pad // C
    row_chunks = chunks_total // NW
    deg_chunks = chunks_total // NS
    rows_per_tile = n_pad // NS
    blocks_per_tile = rows_per_tile // C

    mesh = plsc.VectorSubcoreMesh(core_axis_name="c", subcore_axis_name="s")

    out_type = [jax.ShapeDtypeStruct((NC, n_pad, D), jnp.float32)]
    if with_deg:
        out_type.append(jax.ShapeDtypeStruct((NC, n_pad), jnp.float32))

    scratch = [
        pltpu.VMEM((C,), jnp.int32),
        pltpu.VMEM((C,), jnp.int32),
        pltpu.VMEM((C,), jnp.int32),
        pltpu.VMEM((C,), jnp.int32),
        pltpu.VMEM((C, D), jnp.float32),
        pltpu.VMEM((C, D), jnp.float32),
        pltpu.VMEM((C,), jnp.float32),
        pltpu.VMEM((rows_per_tile,), jnp.float32),
        pltpu.VMEM_SHARED((n_pad, D), jnp.float32),
        pltpu.VMEM_SHARED((n_pad,), jnp.float32),
        pltpu.SemaphoreType.DMA,
        pltpu.SemaphoreType.DMA,
        pltpu.SemaphoreType.DMA,
        pltpu.SemaphoreType.DMA,
    ]

    @functools.partial(
        pl.kernel, mesh=mesh, scratch_types=scratch, out_type=out_type)
    def sc_kernel(y_hbm, src_hbm, dst_hbm, *rest):
        if with_deg:
            (p_hbm, iv_hbm,
             src_v, dst_v, src_w, dst_w, rows_v, rows_w, ones_v, inv_v,
             acc_s, deg_s, gsa, gsb, ssa, ssb) = rest
        else:
            (inv_hbm, p_hbm,
             src_v, dst_v, src_w, dst_w, rows_v, rows_w, ones_v, inv_v,
             acc_s, deg_s, gsa, gsb, ssa, ssb) = rest
        cid = lax.axis_index("c")
        sid = lax.axis_index("s")
        wid = cid * NS + sid
        my_row0 = sid * rows_per_tile
        zvec = jnp.zeros((16,), jnp.float32)
        ovec = jnp.ones((16,), jnp.float32)

        def zero_rows(i, _):
            for j in range(D // 16):
                rows_v[i, pl.ds(j * 16, 16)] = zvec
            return 0

        lax.fori_loop(0, C, zero_rows, 0)

        def zero_inv(i, _):
            inv_v[pl.ds(i * 16, 16)] = zvec
            return 0

        lax.fori_loop(0, rows_per_tile // 16, zero_inv, 0)

        for j in range(C // 16):
            ones_v[pl.ds(j * 16, 16)] = ovec

        for b in range(blocks_per_tile):
            pltpu.sync_copy(rows_v, acc_s.at[pl.ds(my_row0 + b * C, C)])
        if with_deg:
            pltpu.sync_copy(inv_v, deg_s.at[pl.ds(my_row0, rows_per_tile)])
        else:
            pltpu.sync_copy(inv_hbm.at[pl.ds(my_row0, rows_per_tile)],
                            inv_v)

        plsc.subcore_barrier()

        if with_deg:
            def deg_body(k, _):
                base = (sid * deg_chunks + k) * C
                pltpu.sync_copy(dst_hbm.at[pl.ds(base, C)], dst_v)
                pltpu.sync_copy(ones_v, deg_s.at[dst_v], add=True)
                return 0

            lax.fori_loop(0, deg_chunks, deg_body, 0)

        def pair_body(k2, _):
            b0 = (wid * row_chunks + 2 * k2) * C
            b1 = b0 + C
            pltpu.sync_copy(src_hbm.at[pl.ds(b0, C)], src_v)
            pltpu.sync_copy(dst_hbm.at[pl.ds(b0, C)], dst_v)
            ga = pltpu.async_copy(y_hbm.at[src_v], rows_v, gsa)
            pltpu.sync_copy(src_hbm.at[pl.ds(b1, C)], src_w)
            pltpu.sync_copy(dst_hbm.at[pl.ds(b1, C)], dst_w)
            gb = pltpu.async_copy(y_hbm.at[src_w], rows_w, gsb)
            ga.wait()
            sa = pltpu.async_copy(rows_v, acc_s.at[dst_v], ssa, add=True)
            gb.wait()
            sb = pltpu.async_copy(rows_w, acc_s.at[dst_w], ssb, add=True)
            sa.wait()
            sb.wait()
            return 0

        lax.fori_loop(0, row_chunks // 2, pair_body, 0)

        plsc.subcore_barrier()

        if with_deg:
            pltpu.sync_copy(deg_s.at[pl.ds(my_row0, rows_per_tile)], inv_v)

            def invert(i, _):
                d = inv_v[pl.ds(i * 16, 16)]
                inv_v[pl.ds(i * 16, 16)] = ovec / jnp.maximum(d, ovec)
                return 0

            lax.fori_loop(0, rows_per_tile // 16, invert, 0)
            pltpu.sync_copy(inv_v,
                            iv_hbm.at[cid, pl.ds(my_row0, rows_per_tile)])

        for b in range(blocks_per_tile):
            r0 = my_row0 + b * C
            pltpu.sync_copy(acc_s.at[pl.ds(r0, C)], rows_v)

            def scale16(g, _):
                iv = inv_v[pl.ds(b * C + g * 16, 16)]
                for k in range(16):
                    s = zvec + iv[k]
                    row = g * 16 + k
                    for j in range(D // 16):
                        rows_v[row, pl.ds(j * 16, 16)] = (
                            rows_v[row, pl.ds(j * 16, 16)] * s)
                return 0

            lax.fori_loop(0, C // 16, scale16, 0)
            pltpu.sync_copy(rows_v, p_hbm.at[cid, pl.ds(r0, C)])

    return sc_kernel


def kernel(features, edge_index, W_self1, W_neigh1, b1, W_self2, W_neigh2,
           b2):
    n = features.shape[0]
    e = edge_index.shape[1]

    n_pad = -(-(n + 1) // (NS * C)) * (NS * C)
    e_pad = -(-e // (NW * C * 8)) * (NW * C * 8)

    src = edge_index[0]
    dst = edge_index[1]
    if e_pad > e:
        pad = e_pad - e
        ar = jnp.arange(pad, dtype=jnp.int32)
        src = jnp.concatenate([src, ar % n])
        dst = jnp.concatenate([dst, n + ar % (n_pad - n)])

    block_rows = 2000

    sc_agg_deg = _make_sc_aggregate(n_pad, e_pad, True)
    sc_agg = _make_sc_aggregate(n_pad, e_pad, False)

    y1, s1 = _tc_dual_matmul(features, W_neigh1, W_self1, block_rows)
    a1, inv = sc_agg_deg(y1, src, dst)
    y2, s2 = _tc_combine_matmul(s1, a1[0, :n], a1[1, :n], b1, W_neigh2,
                                W_self2, block_rows)
    a2 = sc_agg(y2, src, dst, inv[0])
    if isinstance(a2, (tuple, list)):
        a2 = a2[0]
    out = _tc_final(s2, a2[0, :n], a2[1, :n], b2, block_rows)
    return out
